# baseline (speedup 1.0000x reference)
import functools

import jax
import jax.numpy as jnp
from jax import lax
from jax.experimental import pallas as pl
from jax.experimental.pallas import tpu as pltpu
from jax.experimental.pallas import tpu_sc as plsc

NUM_CORES = 2
NUM_SUBCORES = 16
NUM_WORKERS = NUM_CORES * NUM_SUBCORES

BATCH = 1024
CTX_DIM = 512
PRE_F = 5 * CTX_DIM
CLS_F = 4 * CTX_DIM
SUF_F = 68 * CTX_DIM
ROW_F = PRE_F + CLS_F + SUF_F

ROWS_PER_WORKER = BATCH // NUM_WORKERS
CHUNK = 4


def kernel(label, cls_ctx, token_prefix, token_suffix):
    num_class = cls_ctx.shape[0]
    table = cls_ctx.reshape(num_class, CLS_F)
    pre = token_prefix.reshape(PRE_F)
    suf = token_suffix.reshape(SUF_F)
    lab = label.astype(jnp.int32)

    mesh = plsc.VectorSubcoreMesh(
        core_axis_name="c",
        subcore_axis_name="s",
        num_cores=NUM_CORES,
        num_subcores=NUM_SUBCORES,
    )

    @functools.partial(
        pl.kernel,
        out_type=jax.ShapeDtypeStruct((BATCH, ROW_F), jnp.float32),
        mesh=mesh,
        scratch_types=[
            pltpu.VMEM((ROWS_PER_WORKER,), jnp.int32),
            pltpu.VMEM((ROWS_PER_WORKER, CLS_F), jnp.float32),
            pltpu.VMEM((PRE_F,), jnp.float32),
            pltpu.VMEM((SUF_F,), jnp.float32),
            pltpu.SemaphoreType.DMA,
        ],
    )
    def sc_fill(lab_hbm, table_hbm, pre_hbm, suf_hbm, out_hbm,
                idx_v, cls_v, pre_v, suf_v, sem):
        wid = lax.axis_index("s") * NUM_CORES + lax.axis_index("c")
        base = wid * ROWS_PER_WORKER
        pltpu.sync_copy(lab_hbm.at[pl.ds(base, ROWS_PER_WORKER)], idx_v)
        pltpu.sync_copy(pre_hbm, pre_v)
        pltpu.sync_copy(suf_hbm, suf_v)
        pltpu.async_copy(table_hbm.at[idx_v], cls_v, sem).wait()

        def chunk_body(ci, carry):
            row0 = base + ci * CHUNK
            handles = []
            for j in range(CHUNK):
                row = row0 + j
                r = ci * CHUNK + j
                handles.append(pltpu.async_copy(
                    pre_v, out_hbm.at[row, pl.ds(0, PRE_F)], sem))
                handles.append(pltpu.async_copy(
                    cls_v.at[r], out_hbm.at[row, pl.ds(PRE_F, CLS_F)], sem))
                handles.append(pltpu.async_copy(
                    suf_v, out_hbm.at[row, pl.ds(PRE_F + CLS_F, SUF_F)], sem))
            for h in handles:
                h.wait()
            return carry

        lax.fori_loop(0, ROWS_PER_WORKER // CHUNK, chunk_body, 0)

    out = sc_fill(lab, table, pre, suf)
    return out.reshape(BATCH, 77, CTX_DIM)


# speedup vs baseline: 4.4546x; 4.4546x over previous
import functools

import jax
import jax.numpy as jnp
from jax import lax
from jax.experimental import pallas as pl
from jax.experimental.pallas import tpu as pltpu
from jax.experimental.pallas import tpu_sc as plsc

NUM_CORES = 2
NUM_SUBCORES = 16
NUM_WORKERS = NUM_CORES * NUM_SUBCORES

BATCH = 1024
CTX_DIM = 512
N_PRE = 5
N_CLS = 4
N_SUF = 68
N_TOK = N_PRE + N_CLS + N_SUF
N_HEAD = 16
N_TAIL = N_TOK - N_HEAD

ROWS_PER_WORKER = BATCH // NUM_WORKERS
GATHER_CHUNK = 8

HEAD_BLOCK = 32


def _sc_tails_and_gather(lab, table, tail):
    mesh = plsc.VectorSubcoreMesh(
        core_axis_name="c",
        subcore_axis_name="s",
        num_cores=NUM_CORES,
        num_subcores=NUM_SUBCORES,
    )

    @functools.partial(
        pl.kernel,
        out_type=(
            jax.ShapeDtypeStruct((BATCH, N_TOK, CTX_DIM), jnp.float32),
            jax.ShapeDtypeStruct((BATCH, N_CLS, CTX_DIM), jnp.float32),
        ),
        mesh=mesh,
        scratch_types=[
            pltpu.VMEM((ROWS_PER_WORKER,), jnp.int32),
            pltpu.VMEM((1, N_TAIL, CTX_DIM), jnp.float32),
            pltpu.VMEM((GATHER_CHUNK, N_CLS, CTX_DIM), jnp.float32),
            pltpu.SemaphoreType.DMA,
            pltpu.SemaphoreType.DMA,
            pltpu.SemaphoreType.DMA,
        ],
    )
    def sc_fill(lab_hbm, table_hbm, tail_hbm, out_hbm, cls_hbm,
                idx_v, tail_v, cls_v, gsem, csem, tsem):
        wid = lax.axis_index("s") * NUM_CORES + lax.axis_index("c")
        base = wid * ROWS_PER_WORKER
        pltpu.sync_copy(lab_hbm.at[pl.ds(base, ROWS_PER_WORKER)], idx_v)
        pltpu.sync_copy(tail_hbm, tail_v)

        tail_handles = []
        for r in range(ROWS_PER_WORKER):
            tail_handles.append(pltpu.async_copy(
                tail_v,
                out_hbm.at[pl.ds(base + r, 1), pl.ds(N_HEAD, N_TAIL), :],
                tsem))

        for g in range(ROWS_PER_WORKER // GATHER_CHUNK):
            pltpu.async_copy(
                table_hbm.at[idx_v.at[pl.ds(g * GATHER_CHUNK, GATHER_CHUNK)]],
                cls_v, gsem).wait()
            handles = []
            for j in range(GATHER_CHUNK):
                r = g * GATHER_CHUNK + j
                handles.append(pltpu.async_copy(
                    cls_v.at[pl.ds(j, 1)],
                    cls_hbm.at[pl.ds(base + r, 1)], csem))
            for h in handles:
                h.wait()

        for h in tail_handles:
            h.wait()

    return sc_fill(lab, table, tail)


def _tc_write_heads(cls_all, head_tmpl, tails_out):
    def body(tmpl_ref, cls_ref, _alias_ref, out_ref):
        out_ref[...] = jnp.broadcast_to(tmpl_ref[...],
                                        (HEAD_BLOCK, N_HEAD, CTX_DIM))
        out_ref[:, N_PRE:N_PRE + N_CLS, :] = cls_ref[...]

    return pl.pallas_call(
        body,
        grid=(BATCH // HEAD_BLOCK,),
        in_specs=[
            pl.BlockSpec((1, N_HEAD, CTX_DIM), lambda i: (0, 0, 0)),
            pl.BlockSpec((HEAD_BLOCK, N_CLS, CTX_DIM), lambda i: (i, 0, 0)),
            pl.BlockSpec(memory_space=pltpu.MemorySpace.HBM),
        ],
        out_specs=pl.BlockSpec((HEAD_BLOCK, N_HEAD, CTX_DIM),
                               lambda i: (i, 0, 0)),
        out_shape=jax.ShapeDtypeStruct(tails_out.shape, jnp.float32),
        input_output_aliases={2: 0},
    )(head_tmpl, cls_all, tails_out)


def kernel(label, cls_ctx, token_prefix, token_suffix):
    lab = label.astype(jnp.int32)
    head_tmpl = jnp.concatenate(
        [token_prefix,
         jnp.zeros((1, N_CLS, CTX_DIM), jnp.float32),
         token_suffix[:, :N_HEAD - N_PRE - N_CLS, :]],
        axis=1)
    tail = token_suffix[:, N_HEAD - N_PRE - N_CLS:, :]

    tails, cls_all = _sc_tails_and_gather(lab, cls_ctx, tail)
    return _tc_write_heads(cls_all, head_tmpl, tails)
